# Initial kernel scaffold; baseline (speedup 1.0000x reference)
import jax
import jax.numpy as jnp
from jax.experimental import pallas as pl


def kernel(x_idx, emb_table, w_pad, b_pad):
    raise NotImplementedError("write your pallas kernel here")


# speedup vs baseline: 1.2835x; 1.2835x over previous
import functools

import jax
import jax.numpy as jnp
from jax import lax
from jax.experimental import pallas as pl
from jax.experimental.pallas import tpu as pltpu

O_PAD = 128
TB = 512
ISSUE_UNROLL = 8


def _gather_kernel(idx_ref, emb_hbm, w_ref, b_ref, o_ref, gbuf, sems, *,
                   n_tiles, b_pad):
    t = pl.program_id(0)

    @pl.when(t == 0)
    def _issue_all():
        for tile in range(n_tiles):
            sem = sems.at[tile]

            def body(k, _, tile=tile, sem=sem):
                j0 = tile * TB + k * ISSUE_UNROLL
                for u in range(ISSUE_UNROLL):
                    j = j0 + u
                    g = 2 * j
                    pltpu.make_async_copy(
                        emb_hbm.at[pl.ds(idx_ref[g], 1), :],
                        gbuf.at[pl.ds(j, 1), :],
                        sem).start()
                    pltpu.make_async_copy(
                        emb_hbm.at[pl.ds(idx_ref[g + 1], 1), :],
                        gbuf.at[pl.ds(b_pad + j, 1), :],
                        sem).start()
                return 0

            lax.fori_loop(0, TB // ISSUE_UNROLL, body, 0)

    pltpu.make_async_copy(gbuf.at[pl.ds(0, 2 * TB)],
                          gbuf.at[pl.ds(0, 2 * TB)],
                          sems.at[t]).wait()

    e0 = gbuf[pl.ds(t * TB, TB), :]
    e1 = gbuf[pl.ds(b_pad + t * TB, TB), :]
    h = e0 * e1
    logits = jnp.dot(h, w_ref[...], preferred_element_type=jnp.float32)
    o_ref[...] = jax.nn.sigmoid(logits + b_ref[...])


@jax.jit
def _forward(x_idx, emb_table, w_pad, b_pad):
    B = x_idx.shape[0]
    V, E = emb_table.shape

    B_pad = -(-B // TB) * TB
    n_tiles = B_pad // TB

    idx = jnp.clip(x_idx.astype(jnp.int32), 0, V - 1)
    if B_pad != B:
        idx = jnp.pad(idx, ((0, B_pad - B), (0, 0)))
    idx_flat = idx.reshape(-1)

    kernel_fn = functools.partial(
        _gather_kernel, n_tiles=n_tiles, b_pad=B_pad)

    out = pl.pallas_call(
        kernel_fn,
        out_shape=jax.ShapeDtypeStruct((B_pad, O_PAD), jnp.float32),
        grid_spec=pltpu.PrefetchScalarGridSpec(
            num_scalar_prefetch=1,
            grid=(n_tiles,),
            in_specs=[
                pl.BlockSpec(memory_space=pl.ANY),
                pl.BlockSpec((E, O_PAD), lambda t, idx_s: (0, 0)),
                pl.BlockSpec((1, O_PAD), lambda t, idx_s: (0, 0)),
            ],
            out_specs=pl.BlockSpec((TB, O_PAD), lambda t, idx_s: (t, 0)),
            scratch_shapes=[
                pltpu.VMEM((2 * B_pad, E), jnp.float32),
                pltpu.SemaphoreType.DMA((n_tiles,)),
            ],
        ),
        compiler_params=pltpu.CompilerParams(
            dimension_semantics=("arbitrary",),
            vmem_limit_bytes=int(32 << 20),
            disable_bounds_checks=True,
        ),
    )(idx_flat, emb_table, w_pad, b_pad)

    return out[:B, :2]


def kernel(x_idx, emb_table, w_pad, b_pad):
    return _forward(x_idx, emb_table, w_pad, b_pad)


# speedup vs baseline: 1.2836x; 1.0001x over previous
import functools

import jax
import jax.numpy as jnp
from jax import lax
from jax.experimental import pallas as pl
from jax.experimental.pallas import tpu as pltpu

O_PAD = 128
TB = 512
ISSUE_UNROLL = 16


def _gather_kernel(idx_ref, emb_hbm, w_ref, b_ref, o_ref, gbuf, sems, *,
                   n_tiles, b_pad):
    t = pl.program_id(0)

    @pl.when(t == 0)
    def _issue_all():
        for tile in range(n_tiles):
            sem = sems.at[tile]

            def body(k, _, tile=tile, sem=sem):
                j0 = tile * TB + k * ISSUE_UNROLL
                for u in range(ISSUE_UNROLL):
                    j = j0 + u
                    g = 2 * j
                    pltpu.make_async_copy(
                        emb_hbm.at[pl.ds(idx_ref[g], 1), :],
                        gbuf.at[pl.ds(j, 1), :],
                        sem).start()
                    pltpu.make_async_copy(
                        emb_hbm.at[pl.ds(idx_ref[g + 1], 1), :],
                        gbuf.at[pl.ds(b_pad + j, 1), :],
                        sem).start()
                return 0

            lax.fori_loop(0, TB // ISSUE_UNROLL, body, 0)

    pltpu.make_async_copy(gbuf.at[pl.ds(0, 2 * TB)],
                          gbuf.at[pl.ds(0, 2 * TB)],
                          sems.at[t]).wait()

    e0 = gbuf[pl.ds(t * TB, TB), :]
    e1 = gbuf[pl.ds(b_pad + t * TB, TB), :]
    h = e0 * e1
    logits = jnp.dot(h, w_ref[...], preferred_element_type=jnp.float32)
    o_ref[...] = jax.nn.sigmoid(logits + b_ref[...])


@jax.jit
def _forward(x_idx, emb_table, w_pad, b_pad):
    B = x_idx.shape[0]
    V, E = emb_table.shape

    B_pad = -(-B // TB) * TB
    n_tiles = B_pad // TB

    idx = jnp.clip(x_idx.astype(jnp.int32), 0, V - 1)
    if B_pad != B:
        idx = jnp.pad(idx, ((0, B_pad - B), (0, 0)))
    idx_flat = idx.reshape(-1)

    kernel_fn = functools.partial(
        _gather_kernel, n_tiles=n_tiles, b_pad=B_pad)

    out = pl.pallas_call(
        kernel_fn,
        out_shape=jax.ShapeDtypeStruct((B_pad, O_PAD), jnp.float32),
        grid_spec=pltpu.PrefetchScalarGridSpec(
            num_scalar_prefetch=1,
            grid=(n_tiles,),
            in_specs=[
                pl.BlockSpec(memory_space=pl.ANY),
                pl.BlockSpec((E, O_PAD), lambda t, idx_s: (0, 0)),
                pl.BlockSpec((1, O_PAD), lambda t, idx_s: (0, 0)),
            ],
            out_specs=pl.BlockSpec((TB, O_PAD), lambda t, idx_s: (t, 0)),
            scratch_shapes=[
                pltpu.VMEM((2 * B_pad, E), jnp.float32),
                pltpu.SemaphoreType.DMA((n_tiles,)),
            ],
        ),
        compiler_params=pltpu.CompilerParams(
            dimension_semantics=("arbitrary",),
            vmem_limit_bytes=int(32 << 20),
            disable_bounds_checks=True,
        ),
    )(idx_flat, emb_table, w_pad, b_pad)

    return out[:B, :2]


def kernel(x_idx, emb_table, w_pad, b_pad):
    return _forward(x_idx, emb_table, w_pad, b_pad)


# speedup vs baseline: 1.9090x; 1.4872x over previous
import functools

import jax
import jax.numpy as jnp
from jax import lax
from jax.experimental import pallas as pl
from jax.experimental.pallas import tpu as pltpu

O_PAD = 128
TB = 512
ISSUE_UNROLL = 16


def _gather_kernel(idx_ref, emb_hbm, w_ref, b_ref, o_ref, gbuf, sems, *,
                   n_tiles, b_pad):
    t = pl.program_id(0)

    @pl.when(t == 0)
    def _issue_all():
        for tile in range(n_tiles):
            sem = sems.at[tile]

            def body(k, _, tile=tile, sem=sem):
                j0 = tile * TB + k * ISSUE_UNROLL
                for u in range(ISSUE_UNROLL):
                    j = j0 + u
                    g = 2 * j
                    pltpu.make_async_copy(
                        emb_hbm.at[pl.ds(idx_ref[g], 1), :],
                        gbuf.at[pl.ds(j, 1), :],
                        sem).start()
                    pltpu.make_async_copy(
                        emb_hbm.at[pl.ds(idx_ref[g + 1], 1), :],
                        gbuf.at[pl.ds(b_pad + j, 1), :],
                        sem).start(priority=1)
                return 0

            lax.fori_loop(0, TB // ISSUE_UNROLL, body, 0)

    pltpu.make_async_copy(gbuf.at[pl.ds(0, 2 * TB)],
                          gbuf.at[pl.ds(0, 2 * TB)],
                          sems.at[t]).wait()

    e0 = gbuf[pl.ds(t * TB, TB), :]
    e1 = gbuf[pl.ds(b_pad + t * TB, TB), :]
    h = e0 * e1
    logits = jnp.dot(h, w_ref[...], preferred_element_type=jnp.float32)
    o_ref[...] = jax.nn.sigmoid(logits + b_ref[...])


@jax.jit
def _forward(x_idx, emb_table, w_pad, b_pad):
    B = x_idx.shape[0]
    V, E = emb_table.shape

    B_pad = -(-B // TB) * TB
    n_tiles = B_pad // TB

    idx = jnp.clip(x_idx.astype(jnp.int32), 0, V - 1)
    if B_pad != B:
        idx = jnp.pad(idx, ((0, B_pad - B), (0, 0)))
    idx_flat = idx.reshape(-1)

    kernel_fn = functools.partial(
        _gather_kernel, n_tiles=n_tiles, b_pad=B_pad)

    out = pl.pallas_call(
        kernel_fn,
        out_shape=jax.ShapeDtypeStruct((B_pad, O_PAD), jnp.float32),
        grid_spec=pltpu.PrefetchScalarGridSpec(
            num_scalar_prefetch=1,
            grid=(n_tiles,),
            in_specs=[
                pl.BlockSpec(memory_space=pl.ANY),
                pl.BlockSpec((E, O_PAD), lambda t, idx_s: (0, 0)),
                pl.BlockSpec((1, O_PAD), lambda t, idx_s: (0, 0)),
            ],
            out_specs=pl.BlockSpec((TB, O_PAD), lambda t, idx_s: (t, 0)),
            scratch_shapes=[
                pltpu.VMEM((2 * B_pad, E), jnp.float32),
                pltpu.SemaphoreType.DMA((n_tiles,)),
            ],
        ),
        compiler_params=pltpu.CompilerParams(
            dimension_semantics=("arbitrary",),
            vmem_limit_bytes=int(32 << 20),
            disable_bounds_checks=True,
        ),
    )(idx_flat, emb_table, w_pad, b_pad)

    return out[:B, :2]


def kernel(x_idx, emb_table, w_pad, b_pad):
    return _forward(x_idx, emb_table, w_pad, b_pad)
